# baseline (speedup 1.0000x reference)
import functools

import jax
import jax.numpy as jnp
from jax import lax
from jax.experimental import pallas as pl
from jax.experimental.pallas import tpu as pltpu
from jax.experimental.pallas import tpu_sc as plsc

INT_PO_TYPE = 1
INT_NE_TYPE = 2
D = 16
ROWS = 8
NBUF = 3


def kernel(types, args, table):
    B, L = types.shape
    N = B * L
    t_flat = types.reshape(N)
    a_flat = args.reshape(N)

    info = plsc.get_sparse_core_info()
    NC, NS = info.num_cores, info.num_subcores
    NW = NC * NS
    rows_w = B // NW
    CHUNK = ROWS * L
    assert rows_w * NW == B and rows_w % ROWS == 0 and CHUNK % 16 == 0
    n_ch = rows_w // ROWS

    mesh = plsc.VectorSubcoreMesh(core_axis_name="c", subcore_axis_name="s")

    @functools.partial(
        pl.kernel,
        mesh=mesh,
        out_type=jax.ShapeDtypeStruct((B, L, D), jnp.float32),
        compiler_params=pltpu.CompilerParams(
            use_tc_tiling_on_sc=False, needs_layout_passes=False
        ),
        scratch_types=[
            [pltpu.VMEM((CHUNK,), jnp.int32) for _ in range(NBUF)],
            [pltpu.VMEM((CHUNK,), jnp.float32) for _ in range(NBUF)],
            [pltpu.VMEM((CHUNK, D), jnp.float32) for _ in range(NBUF)],
            [pltpu.SemaphoreType.DMA for _ in range(NBUF)],
            [pltpu.SemaphoreType.DMA for _ in range(NBUF)],
            [pltpu.SemaphoreType.DMA for _ in range(NBUF)],
            [pltpu.SemaphoreType.DMA for _ in range(NBUF)],
        ],
    )
    def emb_kernel(t_hbm, a_hbm, tab_hbm, out_hbm,
                   t_v, a_v, rows_v, tsem, asem, gsem, wsem):
        wid = lax.axis_index("s") * NC + lax.axis_index("c")
        row_w = wid * rows_w
        base_w = row_w * L

        def start_load(ci, s):
            base = base_w + ci * CHUNK
            pltpu.async_copy(t_hbm.at[pl.ds(base, CHUNK)], t_v[s], tsem[s])
            pltpu.async_copy(a_hbm.at[pl.ds(base, CHUNK)], a_v[s], asem[s])

        def wait_load(s):
            pltpu.make_async_copy(t_hbm.at[pl.ds(0, CHUNK)], t_v[s], tsem[s]).wait()
            pltpu.make_async_copy(a_hbm.at[pl.ds(0, CHUNK)], a_v[s], asem[s]).wait()

        def start_gather(s):
            pltpu.async_copy(tab_hbm.at[t_v[s]], rows_v[s], gsem[s])

        def wait_gather(s):
            pltpu.make_async_copy(tab_hbm.at[t_v[s]], rows_v[s], gsem[s]).wait()

        def start_write(ci, s):
            row0 = row_w + ci * ROWS
            for i in range(ROWS):
                pltpu.async_copy(
                    rows_v[s].at[pl.ds(i * L, L)], out_hbm.at[row0 + i], wsem[s])

        def wait_write(s):
            for i in range(ROWS):
                pltpu.make_async_copy(
                    rows_v[s].at[pl.ds(i * L, L)], out_hbm.at[0], wsem[s]).wait()

        def fixup(s):
            rows = rows_v[s]
            tv, av = t_v[s], a_v[s]

            def fix_body(j, c):
                t = tv[pl.ds(j * 16, 16)]
                a = av[pl.ds(j * 16, 16)]
                m = (t == INT_PO_TYPE) | (t == INT_NE_TYPE)
                ridx = j * 16 + lax.iota(jnp.int32, 16)
                cidx = jnp.full((16,), D - 1, jnp.int32)
                plsc.store_scatter(rows, [ridx, cidx], a, mask=m)
                return c

            lax.fori_loop(0, CHUNK // 16, fix_body, 0)

        start_load(0, 0)
        start_load(1, 1)
        wait_load(0)
        start_gather(0)
        for ci in range(n_ch):
            s = ci % NBUF
            if ci + 2 < n_ch:
                s2 = (ci + 2) % NBUF
                if ci >= 1:
                    wait_write(s2)
                start_load(ci + 2, s2)
            if ci + 1 < n_ch:
                s1 = (ci + 1) % NBUF
                wait_load(s1)
                start_gather(s1)
            wait_gather(s)
            fixup(s)
            start_write(ci, s)
        for k in range(min(NBUF, n_ch)):
            wait_write((n_ch - 1 - k) % NBUF)

    return emb_kernel(t_flat, a_flat, table)


# speedup vs baseline: 4.5816x; 4.5816x over previous
import functools

import jax
import jax.numpy as jnp
from jax import lax
from jax.experimental import pallas as pl
from jax.experimental.pallas import tpu as pltpu
from jax.experimental.pallas import tpu_sc as plsc

INT_PO_TYPE = 1
INT_NE_TYPE = 2
D = 16
NBUF = 2


def kernel(types, args, table):
    B, L = types.shape
    V = table.shape[0]
    LH, LL = L // 8, 8
    BH, BL = B // 128, 128
    UK = LL * BL

    t3d = types.reshape(BH, BL, LH, LL).transpose(2, 0, 3, 1).reshape(LH, BH, UK)
    a3d = args.reshape(BH, BL, LH, LL).transpose(2, 0, 3, 1).reshape(LH, BH, UK)
    tabf = table.T.reshape(V * D)

    info = plsc.get_sparse_core_info()
    NC, NS = info.num_cores, info.num_subcores
    NW = NC * NS
    n_units = LH * BH
    units_w = n_units // NW
    assert units_w * NW == n_units and units_w % NBUF == 0 and units_w >= 4

    mesh = plsc.VectorSubcoreMesh(core_axis_name="c", subcore_axis_name="s")

    @functools.partial(
        pl.kernel,
        mesh=mesh,
        out_type=jax.ShapeDtypeStruct((L * 2, BH, UK), jnp.float32),
        compiler_params=pltpu.CompilerParams(
            use_tc_tiling_on_sc=False, needs_layout_passes=False
        ),
        scratch_types=[
            pltpu.VMEM((V * D,), jnp.float32),
            [pltpu.VMEM((UK,), jnp.int32) for _ in range(NBUF)],
            [pltpu.VMEM((UK,), jnp.float32) for _ in range(NBUF)],
            [pltpu.VMEM((D * UK,), jnp.float32) for _ in range(NBUF)],
            [pltpu.SemaphoreType.DMA for _ in range(NBUF)],
            [pltpu.SemaphoreType.DMA for _ in range(NBUF)],
            [pltpu.SemaphoreType.DMA for _ in range(NBUF)],
        ],
    )
    def emb_kernel(t_hbm, a_hbm, tab_hbm, out_hbm,
                   tab_v, t_v, a_v, ob_v, tsem, asem, wsem):
        wid = lax.axis_index("s") * NC + lax.axis_index("c")
        u0 = wid * units_w

        pltpu.sync_copy(tab_hbm, tab_v)

        def start_load(u, s):
            lh, bh = u // BH, u % BH
            pltpu.async_copy(t_hbm.at[lh, bh], t_v[s], tsem[s])
            pltpu.async_copy(a_hbm.at[lh, bh], a_v[s], asem[s])

        def wait_load(s):
            pltpu.make_async_copy(t_hbm.at[0, 0], t_v[s], tsem[s]).wait()
            pltpu.make_async_copy(a_hbm.at[0, 0], a_v[s], asem[s]).wait()

        def start_write(u, s):
            lh, bh = u // BH, u % BH
            for j in range(2 * LL):
                pltpu.async_copy(
                    ob_v[s].at[pl.ds(j * UK, UK)],
                    out_hbm.at[lh * (2 * LL) + j, bh],
                    wsem[s],
                )

        def wait_write(s):
            for j in range(2 * LL):
                pltpu.make_async_copy(
                    ob_v[s].at[pl.ds(j * UK, UK)], out_hbm.at[0, 0], wsem[s]
                ).wait()

        def compute(s):
            tv, av, ob = t_v[s], a_v[s], ob_v[s]

            def grp(j, c):
                t16 = tv[pl.ds(j * 16, 16)]
                a16 = av[pl.ds(j * 16, 16)]
                m = (t16 == INT_PO_TYPE) | (t16 == INT_NE_TYPE)
                base_j = (j // 8) * (2 * UK) + (j % 8) * 16
                for d in range(D):
                    v = plsc.load_gather(tab_v, [t16 + d * V])
                    if d == D - 1:
                        v = jnp.where(m, a16, v)
                    off = base_j + (d // 8) * UK + (d % 8) * BL
                    ob[pl.ds(off, 16)] = v
                return c

            lax.fori_loop(0, UK // 16, grp, 0)

        start_load(u0 + 0, 0)
        start_load(u0 + 1, 1)
        for k in range(2):
            wait_load(k)
            compute(k)
            start_write(u0 + k, k)
            start_load(u0 + k + 2, k)

        def pipe(g, c):
            u = u0 + 2 * g
            for k in range(2):
                wait_write(k)
                wait_load(k)
                compute(k)
                start_write(u + k, k)
                start_load(u + k + 2, k)
            return c

        lax.fori_loop(1, units_w // 2 - 1, pipe, 0)

        u = u0 + units_w - 2
        for k in range(2):
            wait_write(k)
            wait_load(k)
            compute(k)
            start_write(u + k, k)
        wait_write(0)
        wait_write(1)

    out3 = emb_kernel(t3d, a3d, tabf)
    out5 = out3.reshape(L, 2, BH, LL, BL)
    return out5.transpose(2, 4, 0, 1, 3).reshape(B, L, D)


# speedup vs baseline: 12.3519x; 2.6960x over previous
import functools

import jax
import jax.numpy as jnp
from jax import lax
from jax.experimental import pallas as pl
from jax.experimental.pallas import tpu as pltpu
from jax.experimental.pallas import tpu_sc as plsc

INT_PO_TYPE = 1
INT_NE_TYPE = 2
D = 16
NBUF = 2


def kernel(types, args, table):
    B, L = types.shape
    V = table.shape[0]
    LH, LL = L // 8, 8
    BH, BL = B // 128, 128
    UK = LL * BL

    t3d = types.reshape(BH, BL, LH, LL).transpose(2, 0, 3, 1).reshape(LH, BH, UK)
    a3d = args.reshape(BH, BL, LH, LL).transpose(2, 0, 3, 1).reshape(LH, BH, UK)
    tabf = table.T.reshape(V * D)

    info = plsc.get_sparse_core_info()
    NC, NS = info.num_cores, info.num_subcores
    NW = NC * NS
    n_units = LH * BH
    units_w = n_units // NW
    assert units_w * NW == n_units and units_w % NBUF == 0 and units_w >= 4

    mesh = plsc.VectorSubcoreMesh(core_axis_name="c", subcore_axis_name="s")

    @functools.partial(
        pl.kernel,
        mesh=mesh,
        out_type=jax.ShapeDtypeStruct((L * 2, BH, UK), jnp.float32),
        compiler_params=pltpu.CompilerParams(
            use_tc_tiling_on_sc=False, needs_layout_passes=False
        ),
        scratch_types=[
            pltpu.VMEM((V * D,), jnp.float32),
            [pltpu.VMEM((UK,), jnp.int32) for _ in range(NBUF)],
            [pltpu.VMEM((UK,), jnp.float32) for _ in range(NBUF)],
            [pltpu.VMEM((D * UK,), jnp.float32) for _ in range(NBUF)],
            [pltpu.SemaphoreType.DMA for _ in range(NBUF)],
            [pltpu.SemaphoreType.DMA for _ in range(NBUF)],
            [pltpu.SemaphoreType.DMA for _ in range(NBUF)],
        ],
    )
    def emb_kernel(t_hbm, a_hbm, tab_hbm, out_hbm,
                   tab_v, t_v, a_v, ob_v, tsem, asem, wsem):
        wid = lax.axis_index("s") * NC + lax.axis_index("c")
        u0 = wid * units_w

        pltpu.sync_copy(tab_hbm, tab_v)

        def start_load(u, s):
            lh, bh = u // BH, u % BH
            pltpu.async_copy(t_hbm.at[lh, bh], t_v[s], tsem[s])
            pltpu.async_copy(a_hbm.at[lh, bh], a_v[s], asem[s])

        def wait_load(s):
            pltpu.make_async_copy(t_hbm.at[0, 0], t_v[s], tsem[s]).wait()
            pltpu.make_async_copy(a_hbm.at[0, 0], a_v[s], asem[s]).wait()

        def start_write(u, s):
            lh, bh = u // BH, u % BH
            for j in range(2 * LL):
                pltpu.async_copy(
                    ob_v[s].at[pl.ds(j * UK, UK)],
                    out_hbm.at[lh * (2 * LL) + j, bh],
                    wsem[s],
                )

        def wait_write(s):
            for j in range(2 * LL):
                pltpu.make_async_copy(
                    ob_v[s].at[pl.ds(j * UK, UK)], out_hbm.at[0, 0], wsem[s]
                ).wait()

        def compute(s):
            tv, av, ob = t_v[s], a_v[s], ob_v[s]

            @plsc.parallel_loop(0, UK // 16, unroll=4)
            def grp(j):
                t16 = tv[pl.ds(j * 16, 16)]
                a16 = av[pl.ds(j * 16, 16)]
                m = (t16 == INT_PO_TYPE) | (t16 == INT_NE_TYPE)
                base_j = (j // 8) * (2 * UK) + (j % 8) * 16
                for d in range(D):
                    v = plsc.load_gather(tab_v, [t16 + d * V])
                    if d == D - 1:
                        v = jnp.where(m, a16, v)
                    off = base_j + (d // 8) * UK + (d % 8) * BL
                    ob[pl.ds(off, 16)] = v

        start_load(u0 + 0, 0)
        start_load(u0 + 1, 1)
        for k in range(2):
            wait_load(k)
            compute(k)
            start_write(u0 + k, k)
            start_load(u0 + k + 2, k)

        def pipe(g, c):
            u = u0 + 2 * g
            for k in range(2):
                wait_write(k)
                wait_load(k)
                compute(k)
                start_write(u + k, k)
                start_load(u + k + 2, k)
            return c

        lax.fori_loop(1, units_w // 2 - 1, pipe, 0)

        u = u0 + units_w - 2
        for k in range(2):
            wait_write(k)
            wait_load(k)
            compute(k)
            start_write(u + k, k)
        wait_write(0)
        wait_write(1)

    out3 = emb_kernel(t3d, a3d, tabf)
    out5 = out3.reshape(L, 2, BH, LL, BL)
    return out5.transpose(2, 4, 0, 1, 3).reshape(B, L, D)


# speedup vs baseline: 14.0578x; 1.1381x over previous
import functools

import jax
import jax.numpy as jnp
from jax import lax
from jax.experimental import pallas as pl
from jax.experimental.pallas import tpu as pltpu
from jax.experimental.pallas import tpu_sc as plsc

INT_PO_TYPE = 1
INT_NE_TYPE = 2
D = 16
NBUF = 2


def kernel(types, args, table):
    B, L = types.shape
    V = table.shape[0]
    LH, LL = L // 8, 8
    BH, BL = B // 128, 128
    UK = LL * BL

    t3d = types.reshape(BH, BL, LH, LL).transpose(2, 0, 3, 1).reshape(LH, BH, UK)
    a3d = args.reshape(BH, BL, LH, LL).transpose(2, 0, 3, 1).reshape(LH, BH, UK)
    tabf = table.T.reshape(V * D)

    info = plsc.get_sparse_core_info()
    NC, NS = info.num_cores, info.num_subcores
    NW = NC * NS
    n_units = LH * BH
    units_w = n_units // NW
    assert units_w * NW == n_units and units_w % NBUF == 0 and units_w >= 4

    mesh = plsc.VectorSubcoreMesh(core_axis_name="c", subcore_axis_name="s")

    @functools.partial(
        pl.kernel,
        mesh=mesh,
        out_type=jax.ShapeDtypeStruct((L * 2, BH, UK), jnp.float32),
        compiler_params=pltpu.CompilerParams(
            use_tc_tiling_on_sc=False, needs_layout_passes=False
        ),
        scratch_types=[
            pltpu.VMEM((V * D,), jnp.float32),
            [pltpu.VMEM((UK,), jnp.int32) for _ in range(NBUF)],
            [pltpu.VMEM((UK,), jnp.float32) for _ in range(NBUF)],
            [pltpu.VMEM((D * UK,), jnp.float32) for _ in range(NBUF)],
            [pltpu.SemaphoreType.DMA for _ in range(NBUF)],
            [pltpu.SemaphoreType.DMA for _ in range(NBUF)],
            [pltpu.SemaphoreType.DMA for _ in range(NBUF)],
        ],
    )
    def emb_kernel(t_hbm, a_hbm, tab_hbm, out_hbm,
                   tab_v, t_v, a_v, ob_v, tsem, asem, wsem):
        wid = lax.axis_index("s") * NC + lax.axis_index("c")
        u0 = wid * units_w

        pltpu.sync_copy(tab_hbm, tab_v)

        def start_load(u, s):
            lh, bh = u // BH, u % BH
            pltpu.async_copy(t_hbm.at[lh, bh], t_v[s], tsem[s])
            pltpu.async_copy(a_hbm.at[lh, bh], a_v[s], asem[s])

        def wait_load(s):
            pltpu.make_async_copy(t_hbm.at[0, 0], t_v[s], tsem[s]).wait()
            pltpu.make_async_copy(a_hbm.at[0, 0], a_v[s], asem[s]).wait()

        def start_write(u, s):
            lh, bh = u // BH, u % BH
            for j in range(2 * LL):
                pltpu.async_copy(
                    ob_v[s].at[pl.ds(j * UK, UK)],
                    out_hbm.at[lh * (2 * LL) + j, bh],
                    wsem[s],
                )

        def wait_write(s):
            for j in range(2 * LL):
                pltpu.make_async_copy(
                    ob_v[s].at[pl.ds(j * UK, UK)], out_hbm.at[0, 0], wsem[s]
                ).wait()

        def compute(s):
            tv, av, ob = t_v[s], a_v[s], ob_v[s]

            @plsc.parallel_loop(0, UK // 16, unroll=8)
            def grp(j):
                t16 = tv[pl.ds(j * 16, 16)]
                a16 = av[pl.ds(j * 16, 16)]
                m = (t16 == INT_PO_TYPE) | (t16 == INT_NE_TYPE)
                base_j = (j // 8) * (2 * UK) + (j % 8) * 16
                for d in range(D):
                    v = plsc.load_gather(tab_v, [t16 + d * V])
                    if d == D - 1:
                        v = jnp.where(m, a16, v)
                    off = base_j + (d // 8) * UK + (d % 8) * BL
                    ob[pl.ds(off, 16)] = v

        start_load(u0 + 0, 0)
        start_load(u0 + 1, 1)
        for k in range(2):
            wait_load(k)
            compute(k)
            start_write(u0 + k, k)
            start_load(u0 + k + 2, k)

        def pipe(g, c):
            u = u0 + 2 * g
            for k in range(2):
                wait_write(k)
                wait_load(k)
                compute(k)
                start_write(u + k, k)
                start_load(u + k + 2, k)
            return c

        lax.fori_loop(1, units_w // 2 - 1, pipe, 0)

        u = u0 + units_w - 2
        for k in range(2):
            wait_write(k)
            wait_load(k)
            compute(k)
            start_write(u + k, k)
        wait_write(0)
        wait_write(1)

    out3 = emb_kernel(t3d, a3d, tabf)
    out5 = out3.reshape(L, 2, BH, LL, BL)
    return out5.transpose(2, 4, 0, 1, 3).reshape(B, L, D)
